# baseline (speedup 1.0000x reference)
import functools
import jax
import jax.numpy as jnp
from jax import lax
from jax.experimental import pallas as pl
from jax.experimental.pallas import tpu as pltpu
from jax.experimental.pallas import tpu_sc as plsc

DIM = 64
SEQ = 200
BATCH = 4096
NC = 2
NS = 16
NW = NC * NS
NT = BATCH // 128
NTASK = SEQ * NT
TPW = NTASK // NW
NG = DIM // 8
TBL_ROWS = 500000


def _splat(v):
    return jnp.full((16,), v, dtype=jnp.int32)


def _sc_body(xt_hbm, tbl_hbm, pe_hbm, out_hbm,
             pe_v, x0, x1, i0, i1, p0, p1, r0, r1, o0, o1,
             xs0, xs1, gs0, gs1, ss0, ss1):
    xbuf = (x0, x1)
    ibuf = (i0, i1)
    pbuf = (p0, p1)
    rows = (r0, r1)
    obuf = (o0, o1)
    xsem = (xs0, xs1)
    gsem = (gs0, gs1)
    ssem = (ss0, ss1)

    wid = lax.axis_index("s") * NC + lax.axis_index("c")
    t0 = wid * TPW
    pltpu.sync_copy(pe_hbm, pe_v)
    iota = lax.iota(jnp.int32, 16)
    crow = [iota + 16 * m for m in range(8)]

    def task_st(tau):
        return tau >> 5, jnp.bitwise_and(tau, NT - 1)

    def x_desc(tau, p):
        s, t = task_st(tau)
        return pltpu.make_async_copy(
            xt_hbm.at[pl.ds(s, 1), pl.ds(t * 128, 128)], xbuf[p], xsem[p])


    def g_desc(p):
        return pltpu.make_async_copy(tbl_hbm.at[ibuf[p]], rows[p], gsem[p])

    def s_descs(tau, p):
        s, t = task_st(tau)
        return [
            pltpu.make_async_copy(
                obuf[p].at[pl.ds(g * 8, 8)],
                out_hbm.at[s, pl.ds(g * 8, 8), pl.ds(t * 128, 128)],
                ssem[p])
            for g in range(NG)
        ]

    def idxpar(p):
        for j in range(8):
            xv = xbuf[p][0, pl.ds(16 * j, 16)]
            ibuf[p][pl.ds(16 * j, 16)] = lax.shift_right_logical(xv, 1)
            pbuf[p][pl.ds(16 * j, 16)] = lax.shift_left(
                jnp.bitwise_and(xv, 1), 6)

    def compute(tau, p, po):
        s, _ = task_st(tau)
        pe_vecs = [pe_v[s, pl.ds(16 * m, 16)] for m in range(4)]

        @pl.loop(0, 128)
        def _r(r):
            pv = plsc.load_gather(pbuf[p], [jnp.full((16,), r, jnp.int32)])
            r_spl = jnp.full((16,), r, dtype=jnp.int32)
            mlo = pv == 0
            mhi = jnp.logical_not(mlo)
            for m in range(8):
                v = rows[p][r, pl.ds(16 * m, 16)]
                plsc.store_scatter(
                    obuf[po], [crow[m] - pv, r_spl],
                    v * 8.0 + pe_vecs[m % 4],
                    mask=mlo if m < 4 else mhi)

    x_desc(t0, 0).start()
    x_desc(t0 + 1, 1).start()
    x_desc(t0, 0).wait()
    idxpar(0)
    g_desc(0).start()

    @pl.loop(0, TPW // 2)
    def _main(k2):
        for half in range(2):
            tau = t0 + 2 * k2 + half
            rel = 2 * k2 + half
            p = half
            q = 1 - half

            @pl.when(rel < TPW - 1)
            def _():
                x_desc(tau + 1, q).wait()
                idxpar(q)
                g_desc(q).start()

            @pl.when(rel < TPW - 2)
            def _():
                x_desc(tau + 2, p).start()

            g_desc(p).wait()

            @pl.when(rel >= 2)
            def _():
                for d in s_descs(tau - 2, p):
                    d.wait()

            compute(tau, p, p)
            for d in s_descs(tau, p):
                d.start()

    for d in s_descs(t0 + TPW - 2, 0):
        d.wait()
    for d in s_descs(t0 + TPW - 1, 1):
        d.wait()


_mesh = plsc.VectorSubcoreMesh(core_axis_name="c", subcore_axis_name="s")

_pe_call = functools.partial(
    pl.kernel,
    mesh=_mesh,
    out_type=jax.ShapeDtypeStruct((SEQ, DIM, BATCH), jnp.float32),
    scratch_types=[
        pltpu.VMEM((SEQ, 128), jnp.float32),
        pltpu.VMEM((1, 128), jnp.int32),
        pltpu.VMEM((1, 128), jnp.int32),
        pltpu.VMEM((128,), jnp.int32),
        pltpu.VMEM((128,), jnp.int32),
        pltpu.VMEM((144,), jnp.int32),
        pltpu.VMEM((144,), jnp.int32),
        pltpu.VMEM((128, 128), jnp.float32),
        pltpu.VMEM((128, 128), jnp.float32),
        pltpu.VMEM((DIM, 128), jnp.float32),
        pltpu.VMEM((DIM, 128), jnp.float32),
        pltpu.SemaphoreType.DMA,
        pltpu.SemaphoreType.DMA,
        pltpu.SemaphoreType.DMA,
        pltpu.SemaphoreType.DMA,
        pltpu.SemaphoreType.DMA,
        pltpu.SemaphoreType.DMA,
    ],
    compiler_params=pltpu.CompilerParams(
        use_tc_tiling_on_sc=True, needs_layout_passes=False),
)(_sc_body)


@jax.jit
def kernel(x, table, pe):
    xt = x.T
    tbl2 = table.reshape(TBL_ROWS, 128)
    pe_pad = jnp.pad(pe[0, :SEQ, :], ((0, 0), (0, 128 - DIM)))
    out_t = _pe_call(xt, tbl2, pe_pad)
    return out_t.transpose(2, 0, 1)


# speedup vs baseline: 1.4691x; 1.4691x over previous
import functools
import jax
import jax.numpy as jnp
from jax import lax
from jax.experimental import pallas as pl
from jax.experimental.pallas import tpu as pltpu
from jax.experimental.pallas import tpu_sc as plsc

DIM = 64
SEQ = 200
BATCH = 4096
N = BATCH * SEQ
NC = 2
NS = 16
NW = NC * NS
RPW = N // NW
BLK = SEQ
NBLK = RPW // BLK
NBUF = 4
NSL = DIM // 16
G0 = 104
G1 = BLK - G0


def _sc_body(idx_hbm, table_hbm, pe_hbm, out_hbm,
             idx_v, pe_v, r0, r1, r2, r3, o0, o1,
             g0, g1, g2, g3, s0, s1):
    rows = (r0, r1, r2, r3)
    obuf = (o0, o1)
    gsem = (g0, g1, g2, g3)
    ssem = (s0, s1)
    wid = lax.axis_index("s") * NC + lax.axis_index("c")
    base = wid * RPW
    pltpu.sync_copy(idx_hbm.at[pl.ds(base, RPW)], idx_v)
    pltpu.sync_copy(pe_hbm, pe_v)

    def gather_descs(g, buf, sem):
        off = pl.multiple_of(g * BLK, 8)
        return (
            pltpu.make_async_copy(
                table_hbm.at[idx_v.at[pl.ds(off, G0)]], buf.at[pl.ds(0, G0)], sem),
            pltpu.make_async_copy(
                table_hbm.at[idx_v.at[pl.ds(off + G0, G1)]], buf.at[pl.ds(G0, G1)], sem),
        )

    for b in range(NBUF):
        for d in gather_descs(b, rows[b], gsem[b]):
            d.start()

    @pl.loop(0, NBLK // NBUF)
    def _outer(k):
        for b in range(NBUF):
            g = k * NBUF + b
            for d in gather_descs(g, rows[b], gsem[b]):
                d.wait()
            ob = obuf[b % 2]
            osem = ssem[b % 2]

            @pl.when(g >= 2)
            def _():
                pltpu.make_async_copy(ob, out_hbm.at[pl.ds(0, BLK)], osem).wait()

            rb = rows[b]

            @pl.loop(0, BLK, unroll=8)
            def _fma(r):
                for c in range(NSL):
                    sl = pl.ds(c * 16, 16)
                    ob[r, sl] = rb[r, sl] * 8.0 + pe_v[r, sl]

            pltpu.async_copy(ob, out_hbm.at[pl.ds(base + g * BLK, BLK)], osem)

            @pl.when(g + NBUF < NBLK)
            def _():
                for d in gather_descs(g + NBUF, rows[b], gsem[b]):
                    d.start()

    pltpu.make_async_copy(o0, out_hbm.at[pl.ds(0, BLK)], ssem[0]).wait()
    pltpu.make_async_copy(o1, out_hbm.at[pl.ds(0, BLK)], ssem[1]).wait()


_mesh = plsc.VectorSubcoreMesh(core_axis_name="c", subcore_axis_name="s")

_pe_call = functools.partial(
    pl.kernel,
    mesh=_mesh,
    out_type=jax.ShapeDtypeStruct((N, DIM), jnp.float32),
    scratch_types=[
        pltpu.VMEM((RPW,), jnp.int32),
        pltpu.VMEM((SEQ, DIM), jnp.float32),
        pltpu.VMEM((BLK, DIM), jnp.float32),
        pltpu.VMEM((BLK, DIM), jnp.float32),
        pltpu.VMEM((BLK, DIM), jnp.float32),
        pltpu.VMEM((BLK, DIM), jnp.float32),
        pltpu.VMEM((BLK, DIM), jnp.float32),
        pltpu.VMEM((BLK, DIM), jnp.float32),
        pltpu.SemaphoreType.DMA,
        pltpu.SemaphoreType.DMA,
        pltpu.SemaphoreType.DMA,
        pltpu.SemaphoreType.DMA,
        pltpu.SemaphoreType.DMA,
        pltpu.SemaphoreType.DMA,
    ],
    compiler_params=pltpu.CompilerParams(use_tc_tiling_on_sc=False),
)(_sc_body)


@jax.jit
def kernel(x, table, pe):
    idx = x.reshape(-1)
    pe_seq = pe[0, :SEQ, :]
    out = _pe_call(idx, table, pe_seq)
    return out.reshape(x.shape[0], x.shape[1], DIM)
